# Initial kernel scaffold; baseline (speedup 1.0000x reference)
import functools

import jax
import jax.numpy as jnp
from jax import lax
from jax.experimental import pallas as pl
from jax.experimental.pallas import tpu as pltpu
from jax.experimental.pallas import tpu_sc as plsc

N_NODES = 10000
D = 128
N_EDGES = 320000
CHUNK = 128
N_CHUNKS = N_EDGES // CHUNK
NC = 2
NS = 16
NW = NC * NS
N_PAD = 10240
RPT = N_PAD // NS
BLK = 1000

_MESH = plsc.VectorSubcoreMesh(core_axis_name="c", subcore_axis_name="s")


@functools.partial(
    pl.kernel,
    out_type=jax.ShapeDtypeStruct((NC * N_PAD,), jnp.float32),
    mesh=_MESH,
    scratch_types=[
        pltpu.VMEM((CHUNK,), jnp.int32),
        pltpu.VMEM((CHUNK,), jnp.float32),
        pltpu.VMEM((CHUNK,), jnp.float32),
        pltpu.VMEM_SHARED((N_PAD,), jnp.float32),
    ],
)
def _deg_kernel(dst_hbm, ones_hbm, zeros_hbm, deg_hbm, idx_v, ones_v, zeros_v, deg_sp):
    cid = lax.axis_index("c")
    sid = lax.axis_index("s")
    wid = sid * NC + cid
    pltpu.sync_copy(ones_hbm, ones_v)
    pltpu.sync_copy(zeros_hbm, zeros_v)
    for k in range(RPT // CHUNK):
        base = pl.multiple_of(sid * RPT + k * CHUNK, CHUNK)
        pltpu.sync_copy(zeros_v, deg_sp.at[pl.ds(base, CHUNK)])
    plsc.subcore_barrier()

    def body(c):
        base = pl.multiple_of(c * CHUNK, CHUNK)
        pltpu.sync_copy(dst_hbm.at[pl.ds(base, CHUNK)], idx_v)
        pltpu.sync_copy(ones_v, deg_sp.at[idx_v], add=True)
        return c + NW

    lax.while_loop(lambda c: c < N_CHUNKS, body, wid)
    plsc.subcore_barrier()
    src_base = pl.multiple_of(sid * RPT, 8)
    dst_base = pl.multiple_of(cid * N_PAD + sid * RPT, 8)
    pltpu.sync_copy(deg_sp.at[pl.ds(src_base, RPT)], deg_hbm.at[pl.ds(dst_base, RPT)])


@functools.partial(
    pl.kernel,
    out_type=jax.ShapeDtypeStruct((NC * N_PAD, D), jnp.float32),
    mesh=_MESH,
    scratch_types=[
        pltpu.VMEM((CHUNK,), jnp.int32),
        pltpu.VMEM((CHUNK,), jnp.int32),
        pltpu.VMEM((CHUNK, D), jnp.float32),
        pltpu.SemaphoreType.DMA,
        pltpu.VMEM_SHARED((N_PAD, D), jnp.float32),
    ],
)
def _scatter_kernel(y_hbm, src_hbm, dst_hbm, zeros_hbm, out_hbm,
                    src_v, dst_v, rows_v, sem, acc_sp):
    cid = lax.axis_index("c")
    sid = lax.axis_index("s")
    wid = sid * NC + cid
    pltpu.sync_copy(zeros_hbm, rows_v)
    for k in range(RPT // CHUNK):
        base = pl.multiple_of(sid * RPT + k * CHUNK, CHUNK)
        pltpu.sync_copy(rows_v, acc_sp.at[pl.ds(base, CHUNK)])
    plsc.subcore_barrier()

    def body(c):
        base = pl.multiple_of(c * CHUNK, CHUNK)
        pltpu.sync_copy(src_hbm.at[pl.ds(base, CHUNK)], src_v)
        pltpu.sync_copy(dst_hbm.at[pl.ds(base, CHUNK)], dst_v)
        pltpu.async_copy(y_hbm.at[src_v], rows_v, sem).wait()
        pltpu.sync_copy(rows_v, acc_sp.at[dst_v], add=True)
        return c + NW

    lax.while_loop(lambda c: c < N_CHUNKS, body, wid)
    plsc.subcore_barrier()
    pltpu.sync_copy(acc_sp.at[pl.ds(sid * RPT, RPT)],
                    out_hbm.at[pl.ds(cid * N_PAD + sid * RPT, RPT)])


def _mm1_body(x_ref, w_ref, deg_ref, y_ref, dinv_ref):
    deg = deg_ref[0] + deg_ref[1] + 1.0
    dinv = lax.rsqrt(deg)
    dinv_ref[...] = dinv
    y_ref[...] = jnp.dot(x_ref[...], w_ref[...],
                         preferred_element_type=jnp.float32) * dinv


_mm1_call = pl.pallas_call(
    _mm1_body,
    grid=(N_NODES // BLK,),
    in_specs=[
        pl.BlockSpec((BLK, D), lambda i: (i, 0)),
        pl.BlockSpec((D, D), lambda i: (0, 0)),
        pl.BlockSpec((2, BLK, 1), lambda i: (0, i, 0)),
    ],
    out_specs=[
        pl.BlockSpec((BLK, D), lambda i: (i, 0)),
        pl.BlockSpec((BLK, 1), lambda i: (i, 0)),
    ],
    out_shape=[
        jax.ShapeDtypeStruct((N_NODES, D), jnp.float32),
        jax.ShapeDtypeStruct((N_NODES, 1), jnp.float32),
    ],
)


def _layer_body(agg_ref, y1_ref, dinv_ref, b_ref, p_ref, pm_ref, w_ref, y2_ref):
    dinv = dinv_ref[...]
    a = (agg_ref[0] + agg_ref[1] + y1_ref[...]) * dinv
    h = jnp.maximum(a + b_ref[...] + p_ref[...] * pm_ref[0, 0], 0.0)
    y2_ref[...] = jnp.dot(h, w_ref[...],
                          preferred_element_type=jnp.float32) * dinv


_layer_call = pl.pallas_call(
    _layer_body,
    grid=(N_NODES // BLK,),
    in_specs=[
        pl.BlockSpec((2, BLK, D), lambda i: (0, i, 0)),
        pl.BlockSpec((BLK, D), lambda i: (i, 0)),
        pl.BlockSpec((BLK, 1), lambda i: (i, 0)),
        pl.BlockSpec((1, D), lambda i: (0, 0)),
        pl.BlockSpec((BLK, D), lambda i: (i, 0)),
        pl.BlockSpec(memory_space=pltpu.SMEM),
        pl.BlockSpec((D, D), lambda i: (0, 0)),
    ],
    out_specs=pl.BlockSpec((BLK, D), lambda i: (i, 0)),
    out_shape=jax.ShapeDtypeStruct((N_NODES, D), jnp.float32),
)


def _final_body(agg_ref, y2_ref, dinv_ref, b_ref, p_ref, pm_ref, o_ref):
    a = (agg_ref[0] + agg_ref[1] + y2_ref[...]) * dinv_ref[...]
    o_ref[...] = a + b_ref[...] + p_ref[...] * pm_ref[0, 0]


_final_call = pl.pallas_call(
    _final_body,
    grid=(N_NODES // BLK,),
    in_specs=[
        pl.BlockSpec((2, BLK, D), lambda i: (0, i, 0)),
        pl.BlockSpec((BLK, D), lambda i: (i, 0)),
        pl.BlockSpec((BLK, 1), lambda i: (i, 0)),
        pl.BlockSpec((1, D), lambda i: (0, 0)),
        pl.BlockSpec((BLK, D), lambda i: (i, 0)),
        pl.BlockSpec(memory_space=pltpu.SMEM),
    ],
    out_specs=pl.BlockSpec((BLK, D), lambda i: (i, 0)),
    out_shape=jax.ShapeDtypeStruct((N_NODES, D), jnp.float32),
)


def kernel(x, edge_index, W1, b1, W2, b2, P1, P2, cache_name, perturb):
    ei = edge_index.astype(jnp.int32)
    src = ei[0]
    dst = ei[1]
    ones_vec = jnp.ones((CHUNK,), jnp.float32)
    zeros_vec = jnp.zeros((CHUNK,), jnp.float32)
    zeros_mat = jnp.zeros((CHUNK, D), jnp.float32)
    pmask = jnp.where(jnp.asarray(perturb) != 0, 1.0, 0.0).astype(jnp.float32)
    pmask = pmask.reshape(1, 1)

    deg3 = _deg_kernel(dst, ones_vec, zeros_vec).reshape(NC, N_PAD, 1)
    y1, dinv = _mm1_call(x, W1, deg3)
    agg1 = _scatter_kernel(y1, src, dst, zeros_mat).reshape(NC, N_PAD, D)
    y2 = _layer_call(agg1, y1, dinv, b1.reshape(1, D), P1, pmask, W2)
    agg2 = _scatter_kernel(y2, src, dst, zeros_mat).reshape(NC, N_PAD, D)
    return _final_call(agg2, y2, dinv, b2.reshape(1, D), P2, pmask)


# speedup vs baseline: 16.0267x; 16.0267x over previous
import functools

import jax
import jax.numpy as jnp
from jax import lax
from jax.experimental import pallas as pl
from jax.experimental.pallas import tpu as pltpu
from jax.experimental.pallas import tpu_sc as plsc

N_NODES = 10000
D = 128
N_EDGES = 320000
CHUNK = 128
N_CHUNKS = N_EDGES // CHUNK
NC = 2
NS = 16
NW = NC * NS
N_PAD = 10240
RPT = N_PAD // NS
BLK = 1000

_MESH = plsc.VectorSubcoreMesh(core_axis_name="c", subcore_axis_name="s")


@functools.partial(
    pl.kernel,
    out_type=jax.ShapeDtypeStruct((NC * N_PAD,), jnp.float32),
    mesh=_MESH,
    scratch_types=[
        pltpu.VMEM((CHUNK,), jnp.int32),
        pltpu.VMEM((CHUNK,), jnp.float32),
        pltpu.VMEM((CHUNK,), jnp.float32),
        pltpu.VMEM_SHARED((N_PAD,), jnp.float32),
    ],
)
def _deg_kernel(dst_hbm, ones_hbm, zeros_hbm, deg_hbm, idx_v, ones_v, zeros_v, deg_sp):
    cid = lax.axis_index("c")
    sid = lax.axis_index("s")
    wid = sid * NC + cid
    pltpu.sync_copy(ones_hbm, ones_v)
    pltpu.sync_copy(zeros_hbm, zeros_v)
    for k in range(RPT // CHUNK):
        base = pl.multiple_of(sid * RPT + k * CHUNK, CHUNK)
        pltpu.sync_copy(zeros_v, deg_sp.at[pl.ds(base, CHUNK)])
    plsc.subcore_barrier()

    @pl.loop(wid, N_CHUNKS, step=NW)
    def _(c):
        base = pl.multiple_of(c * CHUNK, CHUNK)
        pltpu.sync_copy(dst_hbm.at[pl.ds(base, CHUNK)], idx_v)
        pltpu.sync_copy(ones_v, deg_sp.at[idx_v], add=True)
    plsc.subcore_barrier()
    src_base = pl.multiple_of(sid * RPT, 8)
    dst_base = pl.multiple_of(cid * N_PAD + sid * RPT, 8)
    pltpu.sync_copy(deg_sp.at[pl.ds(src_base, RPT)], deg_hbm.at[pl.ds(dst_base, RPT)])


@functools.partial(
    pl.kernel,
    out_type=jax.ShapeDtypeStruct((NC * N_PAD, D), jnp.float32),
    mesh=_MESH,
    scratch_types=[
        pltpu.VMEM((CHUNK,), jnp.int32),
        pltpu.VMEM((CHUNK,), jnp.int32),
        pltpu.VMEM((CHUNK, D), jnp.float32),
        pltpu.SemaphoreType.DMA,
        pltpu.VMEM_SHARED((N_PAD, D), jnp.float32),
    ],
)
def _scatter_kernel(y_hbm, src_hbm, dst_hbm, zeros_hbm, out_hbm,
                    src_v, dst_v, rows_v, sem, acc_sp):
    cid = lax.axis_index("c")
    sid = lax.axis_index("s")
    wid = sid * NC + cid
    pltpu.sync_copy(zeros_hbm, rows_v)
    for k in range(RPT // CHUNK):
        base = pl.multiple_of(sid * RPT + k * CHUNK, CHUNK)
        pltpu.sync_copy(rows_v, acc_sp.at[pl.ds(base, CHUNK)])
    plsc.subcore_barrier()

    @pl.loop(wid, N_CHUNKS, step=NW)
    def _(c):
        base = pl.multiple_of(c * CHUNK, CHUNK)
        pltpu.sync_copy(src_hbm.at[pl.ds(base, CHUNK)], src_v)
        pltpu.sync_copy(dst_hbm.at[pl.ds(base, CHUNK)], dst_v)
        pltpu.async_copy(y_hbm.at[src_v], rows_v, sem).wait()
        pltpu.sync_copy(rows_v, acc_sp.at[dst_v], add=True)
    plsc.subcore_barrier()
    pltpu.sync_copy(acc_sp.at[pl.ds(sid * RPT, RPT)],
                    out_hbm.at[pl.ds(cid * N_PAD + sid * RPT, RPT)])


def _mm1_body(x_ref, w_ref, deg_ref, y_ref, dinv_ref):
    deg = deg_ref[0] + deg_ref[1] + 1.0
    dinv = lax.rsqrt(deg)
    dinv_ref[...] = dinv
    y_ref[...] = jnp.dot(x_ref[...], w_ref[...],
                         preferred_element_type=jnp.float32) * dinv


_mm1_call = pl.pallas_call(
    _mm1_body,
    grid=(N_NODES // BLK,),
    in_specs=[
        pl.BlockSpec((BLK, D), lambda i: (i, 0)),
        pl.BlockSpec((D, D), lambda i: (0, 0)),
        pl.BlockSpec((2, BLK, 1), lambda i: (0, i, 0)),
    ],
    out_specs=[
        pl.BlockSpec((BLK, D), lambda i: (i, 0)),
        pl.BlockSpec((BLK, 1), lambda i: (i, 0)),
    ],
    out_shape=[
        jax.ShapeDtypeStruct((N_NODES, D), jnp.float32),
        jax.ShapeDtypeStruct((N_NODES, 1), jnp.float32),
    ],
)


def _layer_body(agg_ref, y1_ref, dinv_ref, b_ref, p_ref, pm_ref, w_ref, y2_ref):
    dinv = dinv_ref[...]
    a = (agg_ref[0] + agg_ref[1] + y1_ref[...]) * dinv
    h = jnp.maximum(a + b_ref[...] + p_ref[...] * pm_ref[0, 0], 0.0)
    y2_ref[...] = jnp.dot(h, w_ref[...],
                          preferred_element_type=jnp.float32) * dinv


_layer_call = pl.pallas_call(
    _layer_body,
    grid=(N_NODES // BLK,),
    in_specs=[
        pl.BlockSpec((2, BLK, D), lambda i: (0, i, 0)),
        pl.BlockSpec((BLK, D), lambda i: (i, 0)),
        pl.BlockSpec((BLK, 1), lambda i: (i, 0)),
        pl.BlockSpec((1, D), lambda i: (0, 0)),
        pl.BlockSpec((BLK, D), lambda i: (i, 0)),
        pl.BlockSpec(memory_space=pltpu.SMEM),
        pl.BlockSpec((D, D), lambda i: (0, 0)),
    ],
    out_specs=pl.BlockSpec((BLK, D), lambda i: (i, 0)),
    out_shape=jax.ShapeDtypeStruct((N_NODES, D), jnp.float32),
)


def _final_body(agg_ref, y2_ref, dinv_ref, b_ref, p_ref, pm_ref, o_ref):
    a = (agg_ref[0] + agg_ref[1] + y2_ref[...]) * dinv_ref[...]
    o_ref[...] = a + b_ref[...] + p_ref[...] * pm_ref[0, 0]


_final_call = pl.pallas_call(
    _final_body,
    grid=(N_NODES // BLK,),
    in_specs=[
        pl.BlockSpec((2, BLK, D), lambda i: (0, i, 0)),
        pl.BlockSpec((BLK, D), lambda i: (i, 0)),
        pl.BlockSpec((BLK, 1), lambda i: (i, 0)),
        pl.BlockSpec((1, D), lambda i: (0, 0)),
        pl.BlockSpec((BLK, D), lambda i: (i, 0)),
        pl.BlockSpec(memory_space=pltpu.SMEM),
    ],
    out_specs=pl.BlockSpec((BLK, D), lambda i: (i, 0)),
    out_shape=jax.ShapeDtypeStruct((N_NODES, D), jnp.float32),
)


def kernel(x, edge_index, W1, b1, W2, b2, P1, P2, cache_name, perturb):
    ei = edge_index.astype(jnp.int32)
    src = ei[0]
    dst = ei[1]
    ones_vec = jnp.ones((CHUNK,), jnp.float32)
    zeros_vec = jnp.zeros((CHUNK,), jnp.float32)
    zeros_mat = jnp.zeros((CHUNK, D), jnp.float32)
    pmask = jnp.where(jnp.asarray(perturb) != 0, 1.0, 0.0).astype(jnp.float32)
    pmask = pmask.reshape(1, 1)

    deg3 = _deg_kernel(dst, ones_vec, zeros_vec).reshape(NC, N_PAD, 1)
    y1, dinv = _mm1_call(x, W1, deg3)
    agg1 = _scatter_kernel(y1, src, dst, zeros_mat).reshape(NC, N_PAD, D)
    y2 = _layer_call(agg1, y1, dinv, b1.reshape(1, D), P1, pmask, W2)
    agg2 = _scatter_kernel(y2, src, dst, zeros_mat).reshape(NC, N_PAD, D)
    return _final_call(agg2, y2, dinv, b2.reshape(1, D), P2, pmask)
